# baseline (speedup 1.0000x reference)
import functools

import jax
import jax.numpy as jnp
from jax import lax
from jax.experimental import pallas as pl
from jax.experimental.pallas import tpu as pltpu
from jax.experimental.pallas import tpu_sc as plsc

_NC = 1
_NS = 16
_NW = _NC * _NS


def _gather_body(b_per_w, table, idx, out, idx_v, rows_v, sem):
    wid = lax.axis_index("s") * _NC + lax.axis_index("c")
    base = wid * b_per_w
    pltpu.sync_copy(idx.at[pl.ds(base, b_per_w)], idx_v)
    pltpu.async_copy(table.at[idx_v], rows_v, sem).wait()
    pltpu.sync_copy(rows_v, out.at[pl.ds(base, b_per_w)])


def _sc_gather(table, idx):
    n, d = idx.shape[0], table.shape[1]
    b_per_w = n // _NW
    mesh = plsc.VectorSubcoreMesh(core_axis_name="c", subcore_axis_name="s",
                                  num_cores=_NC)
    return pl.kernel(
        functools.partial(_gather_body, b_per_w),
        out_type=jax.ShapeDtypeStruct((n, d), table.dtype),
        mesh=mesh,
        scratch_types=[
            pltpu.VMEM((b_per_w,), jnp.int32),
            pltpu.VMEM((b_per_w, d), table.dtype),
            pltpu.SemaphoreType.DMA,
        ],
    )(table, idx)


def _sum_body(x_ref, sw_ref, s_ref):
    x = x_ref[...]
    sw = sw_ref[...]
    x2 = jnp.sum(x * x, axis=1, keepdims=True)
    sw2 = jnp.sum(sw * sw, axis=1, keepdims=True)
    g = lax.dot_general(x, sw, (((1,), (1,)), ((), ())),
                        preferred_element_type=jnp.float32)
    d2 = x2 + jnp.transpose(sw2) - 2.0 * g
    dist = jnp.sqrt(jnp.maximum(d2, 0.0))
    s_ref[...] = jnp.sum(jnp.exp(dist), axis=1)


def _final_body(x_ref, tw_ref, s_ref, out_ref):
    diff = x_ref[...] - tw_ref[...]
    td2 = jnp.sum(diff * diff, axis=1)
    out_ref[...] = jnp.sqrt(td2) - jnp.log(s_ref[...])


def kernel(inputs, labels, sample_ids, weight):
    b = labels.shape[0]
    sw = _sc_gather(weight, sample_ids.astype(jnp.int32))
    tw = _sc_gather(weight, labels.astype(jnp.int32))
    s = pl.pallas_call(
        _sum_body,
        out_shape=jax.ShapeDtypeStruct((b,), jnp.float32),
    )(inputs, sw)
    return pl.pallas_call(
        _final_body,
        out_shape=jax.ShapeDtypeStruct((b,), jnp.float32),
    )(inputs, tw, s)


# speedup vs baseline: 1.1229x; 1.1229x over previous
import functools

import jax
import jax.numpy as jnp
from jax import lax
from jax.experimental import pallas as pl
from jax.experimental.pallas import tpu as pltpu
from jax.experimental.pallas import tpu_sc as plsc

_NC = 1
_NS = 16
_NW = _NC * _NS


def _gather_body(b_per_w, b, table, labels, samples, out, idx_v, rows_v, sem):
    wid = lax.axis_index("s") * _NC + lax.axis_index("c")
    base = wid * b_per_w

    @pl.when(base < b)
    def _():
        pltpu.sync_copy(labels.at[pl.ds(base, b_per_w)], idx_v)

    @pl.when(base >= b)
    def _():
        pltpu.sync_copy(samples.at[pl.ds(base - b, b_per_w)], idx_v)

    pltpu.async_copy(table.at[idx_v], rows_v, sem).wait()
    pltpu.sync_copy(rows_v, out.at[pl.ds(base, b_per_w)])


def _sc_gather(table, labels, samples):
    b, s, d = labels.shape[0], samples.shape[0], table.shape[1]
    b_per_w = (b + s) // _NW
    mesh = plsc.VectorSubcoreMesh(core_axis_name="c", subcore_axis_name="s",
                                  num_cores=_NC)
    return pl.kernel(
        functools.partial(_gather_body, b_per_w, b),
        out_type=jax.ShapeDtypeStruct((b + s, d), table.dtype),
        mesh=mesh,
        scratch_types=[
            pltpu.VMEM((b_per_w,), jnp.int32),
            pltpu.VMEM((b_per_w, d), table.dtype),
            pltpu.SemaphoreType.DMA,
        ],
    )(table, labels, samples)


def _rowsum(m):
    ones = jnp.ones((m.shape[1], 1), jnp.float32)
    return lax.dot_general(m, ones, (((1,), (0,)), ((), ())),
                           preferred_element_type=jnp.float32)


def _dense_body(b, x_ref, rows_ref, out_ref):
    x = x_ref[...]
    tw = rows_ref[:b, :]
    sw = rows_ref[b:, :]
    x2 = _rowsum(x * x)
    sw2 = _rowsum(sw * sw)
    g = lax.dot_general(x, sw, (((1,), (1,)), ((), ())),
                        preferred_element_type=jnp.float32)
    d2 = x2 + jnp.transpose(sw2) - 2.0 * g
    dist = jnp.sqrt(jnp.maximum(d2, 0.0))
    s = _rowsum(jnp.exp(dist))
    diff = x - tw
    td2 = _rowsum(diff * diff)
    out_ref[...] = lax.squeeze(jnp.sqrt(td2) - jnp.log(s), (1,))


def _dense(inputs, rows):
    b = inputs.shape[0]
    return pl.pallas_call(
        functools.partial(_dense_body, b),
        out_shape=jax.ShapeDtypeStruct((b,), jnp.float32),
    )(inputs, rows)


def kernel(inputs, labels, sample_ids, weight):
    rows = _sc_gather(weight, labels.astype(jnp.int32),
                      sample_ids.astype(jnp.int32))
    return _dense(inputs, rows)


# speedup vs baseline: 1.1465x; 1.0210x over previous
import functools

import jax
import jax.numpy as jnp
from jax import lax
from jax.experimental import pallas as pl
from jax.experimental.pallas import tpu as pltpu
from jax.experimental.pallas import tpu_sc as plsc

_NC = 1
_NS = 16
_NW = _NC * _NS


def _gather_body(b_per_w, b, table, labels, samples, out, idx_v, rows_v, sem):
    wid = lax.axis_index("s") * _NC + lax.axis_index("c")
    base = wid * b_per_w

    @pl.when(base < b)
    def _():
        pltpu.sync_copy(labels.at[pl.ds(base, b_per_w)], idx_v)

    @pl.when(base >= b)
    def _():
        pltpu.sync_copy(samples.at[pl.ds(base - b, b_per_w)], idx_v)

    pltpu.async_copy(table.at[idx_v], rows_v, sem).wait()
    pltpu.sync_copy(rows_v, out.at[pl.ds(base, b_per_w)])


def _sc_gather(table, labels, samples):
    b, s, d = labels.shape[0], samples.shape[0], table.shape[1]
    b_per_w = (b + s) // _NW
    mesh = plsc.VectorSubcoreMesh(core_axis_name="c", subcore_axis_name="s",
                                  num_cores=_NC)
    return pl.kernel(
        functools.partial(_gather_body, b_per_w, b),
        out_type=jax.ShapeDtypeStruct((b + s, d), table.dtype),
        mesh=mesh,
        scratch_types=[
            pltpu.VMEM((b_per_w,), jnp.int32),
            pltpu.VMEM((b_per_w, d), table.dtype),
            pltpu.SemaphoreType.DMA,
        ],
    )(table, labels, samples)


def _rowsum(m):
    ones = jnp.ones((m.shape[1], 1), jnp.float32)
    return lax.dot_general(m, ones, (((1,), (0,)), ((), ())),
                           preferred_element_type=jnp.float32)


_LOG2E = 1.4426950408889634


def _dense_body(b, x_ref, rows_ref, out_ref):
    x = x_ref[...]
    tw = rows_ref[:b, :]
    sw = rows_ref[b:, :]
    xs = x * _LOG2E
    sws = sw * _LOG2E
    x2 = _rowsum(xs * xs)
    sw2 = _rowsum(sws * sws)
    g = lax.dot_general(xs, sws, (((1,), (1,)), ((), ())),
                        preferred_element_type=jnp.float32)
    m = jnp.maximum(x2 + jnp.transpose(sw2) - 2.0 * g, 0.0)
    dist = m * lax.rsqrt(m + 1e-30)
    s = _rowsum(jnp.exp2(dist))
    diff = x - tw
    td2 = _rowsum(diff * diff)
    out_ref[...] = lax.squeeze(jnp.sqrt(td2) - jnp.log(s), (1,))


def _dense(inputs, rows):
    b = inputs.shape[0]
    return pl.pallas_call(
        functools.partial(_dense_body, b),
        out_shape=jax.ShapeDtypeStruct((b,), jnp.float32),
    )(inputs, rows)


def kernel(inputs, labels, sample_ids, weight):
    rows = _sc_gather(weight, labels.astype(jnp.int32),
                      sample_ids.astype(jnp.int32))
    return _dense(inputs, rows)
